# Initial kernel scaffold; baseline (speedup 1.0000x reference)
import jax
import jax.numpy as jnp
from jax.experimental import pallas as pl


def kernel(activations, dwell_times, transition_counts, dwell_counts, current_dominant, current_dwell):
    raise NotImplementedError("write your pallas kernel here")


# speedup vs baseline: 1.4559x; 1.4559x over previous
import jax
import jax.numpy as jnp
from jax import lax
from jax.experimental import pallas as pl
import jax.experimental.pallas.tpu as pltpu

NS = 4096
MH = 2048
THR = 0.3
R = 256
GRID = NS // R
BIG = 2 ** 30


def _body(sc_ref, act_ref, dc2_ref, dccol_ref, dt_ref, tcin_ref,
          nd_ref, ndw_ref, tocc_ref, mean_ref, tcout_ref, sm):
    i = pl.program_id(0)

    @pl.when(i == 0)
    def _scalars():
        a = act_ref[...]
        mx = jnp.max(a)
        r_io = lax.broadcasted_iota(jnp.int32, (32, 128), 0)
        c_io = lax.broadcasted_iota(jnp.int32, (32, 128), 1)
        lin = r_io * 128 + c_io
        dom = jnp.min(jnp.where(a == mx, lin, BIG))
        is_dom = mx > THR
        prev = sc_ref[0]
        cdw = sc_ref[1]
        prev_valid = prev >= 0
        tocc = is_dom & (dom != prev) & prev_valid
        record_needed = jnp.where(is_dom, tocc, prev_valid)
        safe_prev = jnp.maximum(prev, 0)
        count = jnp.sum(jnp.where(lin == safe_prev, dc2_ref[...], 0))
        can_rec = record_needed & (count < MH)
        new_dom = jnp.where(is_dom, dom, jnp.int32(-1))
        new_dwell = jnp.where(is_dom, jnp.where(tocc, 1, cdw + 1), 0)
        sm[0] = dom
        sm[1] = safe_prev
        sm[2] = tocc.astype(jnp.int32)
        sm[3] = can_rec.astype(jnp.int32)
        sm[4] = cdw
        nd_ref[...] = jnp.full((8, 128), new_dom, jnp.int32)
        ndw_ref[...] = jnp.full((8, 128), new_dwell, jnp.int32)
        tocc_ref[...] = jnp.full((8, 128), tocc.astype(jnp.int32), jnp.int32)

    dom = sm[0]
    safe_prev = sm[1]
    tocc = sm[2]
    can_rec = sm[3]
    cdw = sm[4]
    row0 = i * R

    t = tcin_ref[...]
    hit_tc = (tocc == 1) & (safe_prev >= row0) & (safe_prev < row0 + R)

    @pl.when(hit_tc)
    def _copy_inc():
        rio = lax.broadcasted_iota(jnp.int32, (R, NS), 0) + row0
        cio = lax.broadcasted_iota(jnp.int32, (R, NS), 1)
        tcout_ref[...] = t + jnp.where((rio == safe_prev) & (cio == dom),
                                       jnp.float32(1.0), jnp.float32(0.0))

    @pl.when(jnp.logical_not(hit_tc))
    def _copy():
        tcout_ref[...] = t

    d = dt_ref[...]
    counts = dccol_ref[...]
    rio1 = lax.broadcasted_iota(jnp.int32, (R, 1), 0) + row0
    hit_row = (rio1 == safe_prev) & (can_rec == 1)
    cio2 = lax.broadcasted_iota(jnp.int32, (R, MH), 1)
    m = (cio2 < counts).astype(jnp.float32)
    sums = jnp.sum(d * m, axis=1, keepdims=True)
    sums = sums + jnp.where(hit_row, cdw.astype(jnp.float32), 0.0)
    counts_adj = counts + hit_row.astype(jnp.int32)
    cf = counts_adj.astype(jnp.float32)
    mean_ref[...] = jnp.where(counts_adj > 0,
                              sums / jnp.maximum(cf, 1.0), 0.0)


def kernel(activations, dwell_times, transition_counts, dwell_counts,
           current_dominant, current_dwell):
    act2 = activations.reshape(32, 128)
    dc2 = dwell_counts.reshape(32, 128)
    dccol = dwell_counts.reshape(NS, 1)
    sc = jnp.stack([current_dominant.astype(jnp.int32),
                    current_dwell.astype(jnp.int32)])

    out_shapes = (
        jax.ShapeDtypeStruct((8, 128), jnp.int32),
        jax.ShapeDtypeStruct((8, 128), jnp.int32),
        jax.ShapeDtypeStruct((8, 128), jnp.int32),
        jax.ShapeDtypeStruct((NS, 1), jnp.float32),
        jax.ShapeDtypeStruct((NS, NS), jnp.float32),
    )
    full = lambda shp: pl.BlockSpec(shp, lambda i: (0, 0))
    nd, ndw, tocc, mean, tcounts = pl.pallas_call(
        _body,
        grid=(GRID,),
        in_specs=[
            pl.BlockSpec(memory_space=pltpu.SMEM),
            full((32, 128)),
            full((32, 128)),
            pl.BlockSpec((R, 1), lambda i: (i, 0)),
            pl.BlockSpec((R, MH), lambda i: (i, 0)),
            pl.BlockSpec((R, NS), lambda i: (i, 0)),
        ],
        out_specs=(
            full((8, 128)),
            full((8, 128)),
            full((8, 128)),
            pl.BlockSpec((R, 1), lambda i: (i, 0)),
            pl.BlockSpec((R, NS), lambda i: (i, 0)),
        ),
        out_shape=out_shapes,
        scratch_shapes=[pltpu.SMEM((8,), jnp.int32)],
        compiler_params=pltpu.CompilerParams(
            dimension_semantics=("arbitrary",)),
    )(sc, act2, dc2, dccol, dwell_times, transition_counts)

    return (nd[0, 0].reshape(()),
            ndw[0, 0].reshape(()),
            (tocc[0, 0] != 0).reshape(()),
            mean.reshape(NS),
            tcounts)
